# baseline (speedup 1.0000x reference)
import functools

import jax
import jax.numpy as jnp
from jax import lax
from jax.experimental import pallas as pl
from jax.experimental.pallas import tpu as pltpu
from jax.experimental.pallas import tpu_sc as plsc

_NC = 2
_NS = 16
_L = 16

_ROWS = 8192
_COLS = 2048
_R = 8
_JBLK = _COLS // _L

_RPW0 = 304
_RPW1 = 208


def _permute_body(x_hbm, idx_hbm, out_hbm, idx_v,
                  in0, in1, out0, out1, sin0, sin1, sout0, sout1):
    cid = lax.axis_index("c")
    sid = lax.axis_index("s")
    rpw = _RPW0 - cid * (_RPW0 - _RPW1)
    row_base = cid * (_NS * _RPW0) + sid * rpw
    n_g = (_RPW0 // (2 * _R)) - cid * ((_RPW0 - _RPW1) // (2 * _R))

    ins = (in0, in1)
    outs = (out0, out1)
    sins = (sin0, sin1)
    souts = (sout0, sout1)

    pltpu.sync_copy(idx_hbm, idx_v)

    def in_start(c, b):
        pltpu.async_copy(x_hbm.at[pl.ds(row_base + c * _R, _R)], ins[b],
                         sins[b])

    def in_wait(b):
        pltpu.make_async_copy(x_hbm.at[pl.ds(row_base, _R)], ins[b],
                              sins[b]).wait()

    def out_start(c, b):
        pltpu.async_copy(outs[b], out_hbm.at[pl.ds(row_base + c * _R, _R)],
                         souts[b])

    def out_wait(b):
        pltpu.make_async_copy(outs[b], out_hbm.at[pl.ds(row_base, _R)],
                              souts[b]).wait()

    def compute(b):
        in_b = ins[b]
        out_b = outs[b]

        @plsc.parallel_loop(0, _JBLK, 1, unroll=8)
        def _col(j):
            joff = j * _L
            iv = idx_v[pl.ds(joff, _L)]
            for r in range(_R):
                rv = jnp.full((_L,), r, jnp.int32)
                out_b[r, pl.ds(joff, _L)] = plsc.load_gather(in_b, [rv, iv])

    in_start(0, 0)
    in_start(1, 1)
    for b in (0, 1):
        in_wait(b)
        compute(b)
        out_start(b, b)
        in_start(b + 2, b)

    def g_body(g, carry):
        for b in (0, 1):
            c = 2 * g + b
            in_wait(b)
            out_wait(b)
            compute(b)
            out_start(c, b)
            in_start(c + 2, b)
        return carry

    lax.fori_loop(1, n_g - 1, g_body, 0, unroll=False)

    for b in (0, 1):
        c = 2 * (n_g - 1) + b
        in_wait(b)
        out_wait(b)
        compute(b)
        out_start(c, b)
    out_wait(0)
    out_wait(1)


@functools.partial(jax.jit, static_argnames=())
def kernel(x, idxs):
    x2d = x.reshape(_ROWS, _COLS)
    mesh = plsc.VectorSubcoreMesh(
        core_axis_name="c", subcore_axis_name="s", num_cores=_NC,
        num_subcores=_NS)
    out2d = pl.kernel(
        _permute_body,
        out_type=jax.ShapeDtypeStruct((_ROWS, _COLS), jnp.float32),
        mesh=mesh,
        scratch_types=(
            [pltpu.VMEM((_COLS,), jnp.int32)]
            + [pltpu.VMEM((_R, _COLS), jnp.float32)] * 4
            + [pltpu.SemaphoreType.DMA] * 4
        ),
        compiler_params=pltpu.CompilerParams(needs_layout_passes=False),
    )(x2d, idxs)
    return out2d.reshape(x.shape)


# speedup vs baseline: 1.0119x; 1.0119x over previous
import functools

import jax
import jax.numpy as jnp
from jax import lax
from jax.experimental import pallas as pl
from jax.experimental.pallas import tpu as pltpu
from jax.experimental.pallas import tpu_sc as plsc

_NC = 2
_NS = 16
_L = 16

_ROWS = 8192
_COLS = 2048
_R = 8
_JBLK = _COLS // _L

_RPW0 = 208
_RPW1 = 304


def _permute_body(x_hbm, idx_hbm, out_hbm, idx_v,
                  in0, in1, out0, out1, sin0, sin1, sout0, sout1):
    cid = lax.axis_index("c")
    sid = lax.axis_index("s")
    rpw = _RPW0 - cid * (_RPW0 - _RPW1)
    row_base = cid * (_NS * _RPW0) + sid * rpw
    n_g = (_RPW0 // (2 * _R)) - cid * ((_RPW0 - _RPW1) // (2 * _R))

    ins = (in0, in1)
    outs = (out0, out1)
    sins = (sin0, sin1)
    souts = (sout0, sout1)

    pltpu.sync_copy(idx_hbm, idx_v)

    def in_start(c, b):
        pltpu.async_copy(x_hbm.at[pl.ds(row_base + c * _R, _R)], ins[b],
                         sins[b])

    def in_wait(b):
        pltpu.make_async_copy(x_hbm.at[pl.ds(row_base, _R)], ins[b],
                              sins[b]).wait()

    def out_start(c, b):
        pltpu.async_copy(outs[b], out_hbm.at[pl.ds(row_base + c * _R, _R)],
                         souts[b])

    def out_wait(b):
        pltpu.make_async_copy(outs[b], out_hbm.at[pl.ds(row_base, _R)],
                              souts[b]).wait()

    def compute(b):
        in_b = ins[b]
        out_b = outs[b]

        @plsc.parallel_loop(0, _JBLK, 1, unroll=8)
        def _col(j):
            joff = j * _L
            iv = idx_v[pl.ds(joff, _L)]
            for r in range(_R):
                rv = jnp.full((_L,), r, jnp.int32)
                out_b[r, pl.ds(joff, _L)] = plsc.load_gather(in_b, [rv, iv])

    in_start(0, 0)
    in_start(1, 1)
    for b in (0, 1):
        in_wait(b)
        compute(b)
        out_start(b, b)
        in_start(b + 2, b)

    def g_body(g, carry):
        for b in (0, 1):
            c = 2 * g + b
            in_wait(b)
            out_wait(b)
            compute(b)
            out_start(c, b)
            in_start(c + 2, b)
        return carry

    lax.fori_loop(1, n_g - 1, g_body, 0, unroll=False)

    for b in (0, 1):
        c = 2 * (n_g - 1) + b
        in_wait(b)
        out_wait(b)
        compute(b)
        out_start(c, b)
    out_wait(0)
    out_wait(1)


@functools.partial(jax.jit, static_argnames=())
def kernel(x, idxs):
    x2d = x.reshape(_ROWS, _COLS)
    mesh = plsc.VectorSubcoreMesh(
        core_axis_name="c", subcore_axis_name="s", num_cores=_NC,
        num_subcores=_NS)
    out2d = pl.kernel(
        _permute_body,
        out_type=jax.ShapeDtypeStruct((_ROWS, _COLS), jnp.float32),
        mesh=mesh,
        scratch_types=(
            [pltpu.VMEM((_COLS,), jnp.int32)]
            + [pltpu.VMEM((_R, _COLS), jnp.float32)] * 4
            + [pltpu.SemaphoreType.DMA] * 4
        ),
        compiler_params=pltpu.CompilerParams(needs_layout_passes=False),
    )(x2d, idxs)
    return out2d.reshape(x.shape)


# speedup vs baseline: 1.0728x; 1.0603x over previous
import functools

import jax
import jax.numpy as jnp
from jax import lax
from jax.experimental import pallas as pl
from jax.experimental.pallas import tpu as pltpu
from jax.experimental.pallas import tpu_sc as plsc

_NC = 2
_NS = 16
_NW = _NC * _NS
_L = 16

_ROWS = 8192
_COLS = 2048
_ROWS_PER_W = _ROWS // _NW
_R = 8
_NCHUNK = _ROWS_PER_W // _R
_G = _NCHUNK // 2
_JBLK = _COLS // _L


def _permute_body(x_hbm, idx_hbm, out_hbm, idx_v,
                  in0, in1, out0, out1, sin0, sin1, sout0, sout1):
    wid = lax.axis_index("s") * _NC + lax.axis_index("c")
    row_base = wid * _ROWS_PER_W

    ins = (in0, in1)
    outs = (out0, out1)
    sins = (sin0, sin1)
    souts = (sout0, sout1)

    pltpu.sync_copy(idx_hbm, idx_v)

    def in_start(c, b):
        pltpu.async_copy(x_hbm.at[pl.ds(row_base + c * _R, _R)], ins[b],
                         sins[b])

    def in_wait(b):
        pltpu.make_async_copy(x_hbm.at[pl.ds(row_base, _R)], ins[b],
                              sins[b]).wait()

    def out_start(c, b):
        pltpu.async_copy(outs[b], out_hbm.at[pl.ds(row_base + c * _R, _R)],
                         souts[b])

    def out_wait(b):
        pltpu.make_async_copy(outs[b], out_hbm.at[pl.ds(row_base, _R)],
                              souts[b]).wait()

    def compute(b):
        in_b = ins[b]
        out_b = outs[b]

        @plsc.parallel_loop(0, _JBLK, 1, unroll=8)
        def _col(j):
            joff = j * _L
            iv = idx_v[pl.ds(joff, _L)]
            for r in range(_R):
                rv = jnp.full((_L,), r, jnp.int32)
                out_b[r, pl.ds(joff, _L)] = plsc.load_gather(in_b, [rv, iv])

    in_start(0, 0)
    in_start(1, 1)
    for b in (0, 1):
        in_wait(b)
        compute(b)
        out_start(b, b)
        in_start(b + 2, b)

    def g_body(g, carry):
        for b in (0, 1):
            c = 2 * g + b
            in_wait(b)
            out_wait(b)
            compute(b)
            out_start(c, b)
            in_start(c + 2, b)
        return carry

    lax.fori_loop(1, _G - 1, g_body, 0, unroll=False)

    for b in (0, 1):
        c = 2 * (_G - 1) + b
        in_wait(b)
        out_wait(b)
        compute(b)
        out_start(c, b)
    out_wait(0)
    out_wait(1)


@functools.partial(jax.jit, static_argnames=())
def kernel(x, idxs):
    x2d = x.reshape(_ROWS, _COLS)
    mesh = plsc.VectorSubcoreMesh(
        core_axis_name="c", subcore_axis_name="s", num_cores=_NC,
        num_subcores=_NS)
    out2d = pl.kernel(
        _permute_body,
        out_type=jax.ShapeDtypeStruct((_ROWS, _COLS), jnp.float32),
        mesh=mesh,
        scratch_types=(
            [pltpu.VMEM((_COLS,), jnp.int32)]
            + [pltpu.VMEM((_R, _COLS), jnp.float32)] * 4
            + [pltpu.SemaphoreType.DMA] * 4
        ),
        compiler_params=pltpu.CompilerParams(needs_layout_passes=False),
    )(x2d, idxs)
    return out2d.reshape(x.shape)


# speedup vs baseline: 1.0781x; 1.0049x over previous
import functools

import jax
import jax.numpy as jnp
from jax import lax
from jax.experimental import pallas as pl
from jax.experimental.pallas import tpu as pltpu
from jax.experimental.pallas import tpu_sc as plsc

_NC = 2
_NS = 16
_NW = _NC * _NS
_L = 16

_ROWS = 8192
_COLS = 2048
_ROWS_PER_W = _ROWS // _NW
_R = 8
_NCHUNK = _ROWS_PER_W // _R
_G = _NCHUNK // 2
_JBLK = _COLS // _L


def _permute_body(x_hbm, idx_hbm, out_hbm, idx_v,
                  in0, in1, out0, out1, sin0, sin1, sout0, sout1):
    wid = lax.axis_index("s") * _NC + lax.axis_index("c")
    row_base = wid * _ROWS_PER_W

    ins = (in0, in1)
    outs = (out0, out1)
    sins = (sin0, sin1)
    souts = (sout0, sout1)

    def in_start(c, b):
        pltpu.async_copy(x_hbm.at[pl.ds(row_base + c * _R, _R)], ins[b],
                         sins[b])

    def in_wait(b):
        pltpu.make_async_copy(x_hbm.at[pl.ds(row_base, _R)], ins[b],
                              sins[b]).wait()

    def out_start(c, b):
        pltpu.async_copy(outs[b], out_hbm.at[pl.ds(row_base + c * _R, _R)],
                         souts[b])

    def out_wait(b):
        pltpu.make_async_copy(outs[b], out_hbm.at[pl.ds(row_base, _R)],
                              souts[b]).wait()

    def compute(b):
        in_b = ins[b]
        out_b = outs[b]

        @plsc.parallel_loop(0, _JBLK, 1, unroll=8)
        def _col(j):
            joff = j * _L
            iv = idx_v[pl.ds(joff, _L)]
            for r in range(_R):
                rv = jnp.full((_L,), r, jnp.int32)
                out_b[r, pl.ds(joff, _L)] = plsc.load_gather(in_b, [rv, iv])

    in_start(0, 0)
    in_start(1, 1)
    pltpu.sync_copy(idx_hbm, idx_v)
    for b in (0, 1):
        in_wait(b)
        compute(b)
        out_start(b, b)
        in_start(b + 2, b)

    def g_body(g, carry):
        for b in (0, 1):
            c = 2 * g + b
            in_wait(b)
            out_wait(b)
            compute(b)
            out_start(c, b)
            in_start(c + 2, b)
        return carry

    lax.fori_loop(1, _G - 1, g_body, 0, unroll=False)

    for b in (0, 1):
        c = 2 * (_G - 1) + b
        in_wait(b)
        out_wait(b)
        compute(b)
        out_start(c, b)
    out_wait(0)
    out_wait(1)


@functools.partial(jax.jit, static_argnames=())
def kernel(x, idxs):
    x2d = x.reshape(_ROWS, _COLS)
    mesh = plsc.VectorSubcoreMesh(
        core_axis_name="c", subcore_axis_name="s", num_cores=_NC,
        num_subcores=_NS)
    out2d = pl.kernel(
        _permute_body,
        out_type=jax.ShapeDtypeStruct((_ROWS, _COLS), jnp.float32),
        mesh=mesh,
        scratch_types=(
            [pltpu.VMEM((_COLS,), jnp.int32)]
            + [pltpu.VMEM((_R, _COLS), jnp.float32)] * 4
            + [pltpu.SemaphoreType.DMA] * 4
        ),
        compiler_params=pltpu.CompilerParams(needs_layout_passes=False),
    )(x2d, idxs)
    return out2d.reshape(x.shape)
